# baseline (speedup 1.0000x reference)
import functools

import jax
import jax.numpy as jnp
from jax import lax
from jax.experimental import pallas as pl
from jax.experimental.pallas import tpu as pltpu
from jax.experimental.pallas import tpu_sc as plsc

_NC = 2
_NS = 16
_NW = _NC * _NS


def _mesh():
    return plsc.VectorSubcoreMesh(
        core_axis_name="c",
        subcore_axis_name="s",
        num_cores=_NC,
        num_subcores=_NS,
    )


@functools.lru_cache(maxsize=None)
def _build_transpose(V, D):
    assert D == 32
    PW = 512
    n_full = V // PW
    tail = V % PW
    assert tail % 4 == 0 and tail < PW
    L_rows = V // 4
    NP = (n_full + _NW - 1) // _NW
    assert NP % 2 == 0
    mesh = _mesh()

    @functools.partial(
        pl.kernel,
        out_type=jax.ShapeDtypeStruct((L_rows, 128), jnp.float32),
        mesh=mesh,
        scratch_types=[
            pltpu.VMEM((2, D, PW), jnp.float32),
            pltpu.VMEM((2, PW // 4, 128), jnp.float32),
            pltpu.SemaphoreType.DMA,
            pltpu.SemaphoreType.DMA,
        ],
        compiler_params=pltpu.CompilerParams(needs_layout_passes=False),
    )
    def transpose_table(embT, tail_lines, L, ibuf, obuf, isem, osem):
        wid = lax.axis_index("s") * _NC + lax.axis_index("c")
        d_lo = lax.iota(jnp.int32, 16)
        d_hi = d_lo + 16
        zeros16 = jnp.zeros((16,), jnp.int32)

        def in_copy(p, b):
            return pltpu.make_async_copy(
                embT.at[:, pl.ds(p * PW, PW)], ibuf.at[b], isem
            )

        def out_copy(p, b):
            return pltpu.make_async_copy(
                obuf.at[b], L.at[pl.ds(p * (PW // 4), PW // 4), :], osem
            )

        def do_transpose(b, nlines):
            @pl.loop(0, nlines, unroll=4)
            def _(l):
                for jblk in range(8):
                    dvec = d_lo if jblk % 2 == 0 else d_hi
                    colvec = zeros16 + (4 * l + (jblk // 2))
                    vals = plsc.load_gather(ibuf.at[b], [dvec, colvec])
                    obuf[b, l, pl.ds(jblk * 16, 16)] = vals

        in_copy(wid, 0).start()

        @pl.loop(0, NP, step=2)
        def _(i0):
            for par in range(2):
                i = i0 + par
                pid = wid + _NW * i

                @pl.when(pid < n_full)
                def _():
                    in_copy(pid, par).wait()
                    nxt = pid + _NW

                    @pl.when(nxt < n_full)
                    def _():
                        in_copy(nxt, 1 - par).start()

                    @pl.when(i >= 2)
                    def _():
                        out_copy(wid, par).wait()

                    do_transpose(par, PW // 4)
                    out_copy(pid, par).start()

        out_copy(wid, 0).wait()
        out_copy(wid, 0).wait()

        if tail:
            nt = tail // 4

            @pl.when(wid == _NW - 1)
            def _():
                pltpu.sync_copy(tail_lines, obuf.at[0, pl.ds(0, nt), :])
                pltpu.sync_copy(
                    obuf.at[0, pl.ds(0, nt), :],
                    L.at[pl.ds(L_rows - nt, nt), :],
                )

    return transpose_table


@functools.lru_cache(maxsize=None)
def _build_gather(S, NB, V, D):
    assert D == 32
    CB = 128
    assert NB // CB == _NW and NB % CB == 0
    assert S % 2 == 0
    L_rows = V // 4
    mesh = _mesh()

    @functools.partial(
        pl.kernel,
        out_type=jax.ShapeDtypeStruct((S, D, NB), jnp.float32),
        mesh=mesh,
        scratch_types=[
            pltpu.VMEM((S, CB), jnp.int32),
            pltpu.VMEM((2, CB), jnp.int32),
            pltpu.VMEM((2, CB, 128), jnp.float32),
            pltpu.VMEM((2, D, CB), jnp.float32),
            pltpu.SemaphoreType.DMA,
            pltpu.SemaphoreType.DMA,
        ],
        compiler_params=pltpu.CompilerParams(needs_layout_passes=False),
    )
    def gather_native(tidsT, L, outT, idxall, linebuf, gbuf, obuf, gsem, osem):
        wid = lax.axis_index("s") * _NC + lax.axis_index("c")
        b0 = wid * CB
        pltpu.sync_copy(tidsT.at[:, pl.ds(b0, CB)], idxall)
        rowbase = lax.iota(jnp.int32, 16)

        def prep_lines(s, par):
            for cb in range(8):
                t = idxall[s, pl.ds(cb * 16, 16)]
                linebuf[par, pl.ds(cb * 16, 16)] = jax.lax.shift_right_logical(t, 2)

        def g_copy(par):
            return pltpu.make_async_copy(
                L.at[linebuf.at[par]], gbuf.at[par], gsem
            )

        def o_copy(s, par):
            return pltpu.make_async_copy(
                obuf.at[par], outT.at[s, :, pl.ds(b0, CB)], osem
            )

        def extract(s, par):
            for cb in range(8):
                rowvec = rowbase + (cb * 16)
                t = idxall[s, pl.ds(cb * 16, 16)]
                m32 = (t & 3) * 32

                @pl.loop(0, D, unroll=8)
                def _(d):
                    colvec = m32 + d
                    vals = plsc.load_gather(gbuf.at[par], [rowvec, colvec])
                    obuf[par, d, pl.ds(cb * 16, 16)] = vals

        prep_lines(0, 0)
        g_copy(0).start()

        @pl.loop(0, S, step=2)
        def _(s0):
            for par in range(2):
                s = s0 + par

                @pl.when(s + 1 < S)
                def _():
                    prep_lines(s + 1, 1 - par)
                    g_copy(1 - par).start()

                g_copy(par).wait()

                @pl.when(s >= 2)
                def _():
                    o_copy(0, par).wait()

                extract(s, par)
                o_copy(s, par).start()

        o_copy(0, 0).wait()
        o_copy(0, 0).wait()

    return gather_native


def kernel(token_ids, emb):
    NB, S = token_ids.shape
    V, D = emb.shape
    embT = emb.T
    tidsT = token_ids.T.astype(jnp.int32)
    tail = V % 512
    tail_lines = emb[V - tail:, :].reshape(tail // 4, 128)
    L = _build_transpose(V, D)(embT, tail_lines)
    outT = _build_gather(S, NB, V, D)(tidsT, L)
    return outT.transpose(2, 0, 1)


# speedup vs baseline: 1.3160x; 1.3160x over previous
import functools

import jax
import jax.numpy as jnp
from jax import lax
from jax.experimental import pallas as pl
from jax.experimental.pallas import tpu as pltpu
from jax.experimental.pallas import tpu_sc as plsc

_NC = 2
_NS = 16
_NW = _NC * _NS


def _mesh():
    return plsc.VectorSubcoreMesh(
        core_axis_name="c",
        subcore_axis_name="s",
        num_cores=_NC,
        num_subcores=_NS,
    )


@functools.lru_cache(maxsize=None)
def _build_transpose(V, D):
    assert D == 32
    PW = 512
    n_full = V // PW
    tail = V % PW
    assert tail % 4 == 0 and tail < PW
    L_rows = V // 4
    NP = (n_full + _NW - 1) // _NW
    assert NP % 2 == 0
    mesh = _mesh()

    TS = PW + 1

    @functools.partial(
        pl.kernel,
        out_type=jax.ShapeDtypeStruct((L_rows, 128), jnp.float32),
        mesh=mesh,
        scratch_types=[
            pltpu.VMEM((2, D, PW), jnp.float32),
            pltpu.VMEM((2, PW // 4, 128), jnp.float32),
            pltpu.VMEM((D * TS,), jnp.float32),
            pltpu.SemaphoreType.DMA,
            pltpu.SemaphoreType.DMA,
        ],
        compiler_params=pltpu.CompilerParams(needs_layout_passes=False),
    )
    def transpose_table(embT, tail_lines, L, ibuf, obuf, tmp, isem, osem):
        wid = lax.axis_index("s") * _NC + lax.axis_index("c")
        lane = lax.iota(jnp.int32, 16)
        sd_lo = lane * TS
        sd_hi = sd_lo + 16 * TS

        def in_copy(p, b):
            return pltpu.make_async_copy(
                embT.at[:, pl.ds(p * PW, PW)], ibuf.at[b], isem
            )

        def out_copy(p, b):
            return pltpu.make_async_copy(
                obuf.at[b], L.at[pl.ds(p * (PW // 4), PW // 4), :], osem
            )

        def do_transpose(b, nlines):
            @pl.loop(0, D, unroll=2)
            def _(d):
                base = d * TS
                for blk in range(PW // 16):
                    v = ibuf[b, d, pl.ds(blk * 16, 16)]
                    plsc.store_scatter(tmp, [lane + (base + blk * 16)], v)

            @pl.loop(0, nlines, unroll=4)
            def _(l):
                for jblk in range(8):
                    sd = sd_lo if jblk % 2 == 0 else sd_hi
                    addr = sd + (4 * l + (jblk // 2))
                    vals = plsc.load_gather(tmp, [addr])
                    obuf[b, l, pl.ds(jblk * 16, 16)] = vals

        in_copy(wid, 0).start()

        @pl.loop(0, NP, step=2)
        def _(i0):
            for par in range(2):
                i = i0 + par
                pid = wid + _NW * i

                @pl.when(pid < n_full)
                def _():
                    in_copy(pid, par).wait()
                    nxt = pid + _NW

                    @pl.when(nxt < n_full)
                    def _():
                        in_copy(nxt, 1 - par).start()

                    @pl.when(i >= 2)
                    def _():
                        out_copy(wid, par).wait()

                    do_transpose(par, PW // 4)
                    out_copy(pid, par).start()

        out_copy(wid, 0).wait()
        out_copy(wid, 0).wait()

        if tail:
            nt = tail // 4

            @pl.when(wid == _NW - 1)
            def _():
                pltpu.sync_copy(tail_lines, obuf.at[0, pl.ds(0, nt), :])
                pltpu.sync_copy(
                    obuf.at[0, pl.ds(0, nt), :],
                    L.at[pl.ds(L_rows - nt, nt), :],
                )

    return transpose_table


@functools.lru_cache(maxsize=None)
def _build_gather(S, NB, V, D):
    assert D == 32
    CB = 128
    assert NB // CB == _NW and NB % CB == 0
    assert S % 2 == 0
    L_rows = V // 4
    mesh = _mesh()

    GS = 33

    @functools.partial(
        pl.kernel,
        out_type=jax.ShapeDtypeStruct((S, D, NB), jnp.float32),
        mesh=mesh,
        scratch_types=[
            pltpu.VMEM((S, CB), jnp.int32),
            pltpu.VMEM((2, CB), jnp.int32),
            pltpu.VMEM((2, CB, 128), jnp.float32),
            pltpu.VMEM((2, D, CB), jnp.float32),
            pltpu.VMEM((CB * GS,), jnp.float32),
            pltpu.SemaphoreType.DMA,
            pltpu.SemaphoreType.DMA,
        ],
        compiler_params=pltpu.CompilerParams(needs_layout_passes=False),
    )
    def gather_native(tidsT, L, outT, idxall, linebuf, gbuf, obuf, tmp, gsem, osem):
        wid = lax.axis_index("s") * _NC + lax.axis_index("c")
        b0 = wid * CB
        pltpu.sync_copy(tidsT.at[:, pl.ds(b0, CB)], idxall)
        lane = lax.iota(jnp.int32, 16)
        cv = [GS * (lane + 16 * blk) for blk in range(8)]

        def prep_lines(s, par):
            for cb in range(8):
                t = idxall[s, pl.ds(cb * 16, 16)]
                linebuf[par, pl.ds(cb * 16, 16)] = jax.lax.shift_right_logical(t, 2)

        def g_copy(par):
            return pltpu.make_async_copy(
                L.at[linebuf.at[par]], gbuf.at[par], gsem
            )

        def o_copy(s, par):
            return pltpu.make_async_copy(
                obuf.at[par], outT.at[s, :, pl.ds(b0, CB)], osem
            )

        def extract(s, par):
            for blk in range(8):
                tvec = idxall[s, pl.ds(blk * 16, 16)]
                m32v = (tvec & 3) * 32
                for i in range(16):
                    c = blk * 16 + i
                    col_lo = lane + m32v[i]
                    col_hi = col_lo + 16
                    row = jnp.full((16,), c, jnp.int32)
                    dst_lo = lane + (GS * c)
                    dst_hi = dst_lo + 16
                    v_lo = plsc.load_gather(gbuf.at[par], [row, col_lo])
                    v_hi = plsc.load_gather(gbuf.at[par], [row, col_hi])
                    plsc.store_scatter(tmp, [dst_lo], v_lo)
                    plsc.store_scatter(tmp, [dst_hi], v_hi)

            @pl.loop(0, D, unroll=4)
            def _(d):
                for blk in range(8):
                    vals = plsc.load_gather(tmp, [cv[blk] + d])
                    obuf[par, d, pl.ds(blk * 16, 16)] = vals

        prep_lines(0, 0)
        g_copy(0).start()

        @pl.loop(0, S, step=2)
        def _(s0):
            for par in range(2):
                s = s0 + par

                @pl.when(s + 1 < S)
                def _():
                    prep_lines(s + 1, 1 - par)
                    g_copy(1 - par).start()

                g_copy(par).wait()

                @pl.when(s >= 2)
                def _():
                    o_copy(0, par).wait()

                extract(s, par)
                o_copy(s, par).start()

        o_copy(0, 0).wait()
        o_copy(0, 0).wait()

    return gather_native


def kernel(token_ids, emb):
    NB, S = token_ids.shape
    V, D = emb.shape
    embT = emb.T
    tidsT = token_ids.T.astype(jnp.int32)
    tail = V % 512
    tail_lines = emb[V - tail:, :].reshape(tail // 4, 128)
    L = _build_transpose(V, D)(embT, tail_lines)
    outT = _build_gather(S, NB, V, D)(tidsT, L)
    return outT.transpose(2, 0, 1)


# speedup vs baseline: 4.1887x; 3.1828x over previous
import functools

import jax
import jax.numpy as jnp
from jax import lax
from jax.experimental import pallas as pl
from jax.experimental.pallas import tpu as pltpu
from jax.experimental.pallas import tpu_sc as plsc

_NC = 2
_NS = 16
_NW = _NC * _NS


def _mesh():
    return plsc.VectorSubcoreMesh(
        core_axis_name="c",
        subcore_axis_name="s",
        num_cores=_NC,
        num_subcores=_NS,
    )


@functools.lru_cache(maxsize=None)
def _build_transpose(V, D):
    assert D == 32
    PW = 512
    n_full = V // PW
    tail = V % PW
    assert tail % 4 == 0 and tail < PW
    L_rows = V // 4
    NP = (n_full + _NW - 1) // _NW
    assert NP % 2 == 0
    mesh = _mesh()

    TS = PW + 1

    @functools.partial(
        pl.kernel,
        out_type=jax.ShapeDtypeStruct((L_rows, 128), jnp.float32),
        mesh=mesh,
        scratch_types=[
            pltpu.VMEM((2, D, PW), jnp.float32),
            pltpu.VMEM((2, PW // 4, 128), jnp.float32),
            pltpu.VMEM((D * TS,), jnp.float32),
            pltpu.SemaphoreType.DMA,
            pltpu.SemaphoreType.DMA,
        ],
        compiler_params=pltpu.CompilerParams(needs_layout_passes=False),
    )
    def transpose_table(embT, tail_lines, L, ibuf, obuf, tmp, isem, osem):
        wid = lax.axis_index("s") * _NC + lax.axis_index("c")
        lane = lax.iota(jnp.int32, 16)
        sd_lo = lane * TS
        sd_hi = sd_lo + 16 * TS

        def in_copy(p, b):
            return pltpu.make_async_copy(
                embT.at[:, pl.ds(p * PW, PW)], ibuf.at[b], isem
            )

        def out_copy(p, b):
            return pltpu.make_async_copy(
                obuf.at[b], L.at[pl.ds(p * (PW // 4), PW // 4), :], osem
            )

        def do_transpose(b, nlines):
            @plsc.parallel_loop(0, D, unroll=2)
            def _(d):
                base = d * TS
                for blk in range(PW // 16):
                    v = ibuf[b, d, pl.ds(blk * 16, 16)]
                    plsc.store_scatter(tmp, [lane + (base + blk * 16)], v)

            @plsc.parallel_loop(0, nlines, unroll=4)
            def _(l):
                for jblk in range(8):
                    sd = sd_lo if jblk % 2 == 0 else sd_hi
                    addr = sd + (4 * l + (jblk // 2))
                    vals = plsc.load_gather(tmp, [addr])
                    obuf[b, l, pl.ds(jblk * 16, 16)] = vals

        in_copy(wid, 0).start()

        @pl.loop(0, NP, step=2)
        def _(i0):
            for par in range(2):
                i = i0 + par
                pid = wid + _NW * i

                @pl.when(pid < n_full)
                def _():
                    in_copy(pid, par).wait()
                    nxt = pid + _NW

                    @pl.when(nxt < n_full)
                    def _():
                        in_copy(nxt, 1 - par).start()

                    @pl.when(i >= 2)
                    def _():
                        out_copy(wid, par).wait()

                    do_transpose(par, PW // 4)
                    out_copy(pid, par).start()

        out_copy(wid, 0).wait()
        out_copy(wid, 0).wait()

        if tail:
            nt = tail // 4

            @pl.when(wid == _NW - 1)
            def _():
                pltpu.sync_copy(tail_lines, obuf.at[0, pl.ds(0, nt), :])
                pltpu.sync_copy(
                    obuf.at[0, pl.ds(0, nt), :],
                    L.at[pl.ds(L_rows - nt, nt), :],
                )

    return transpose_table


@functools.lru_cache(maxsize=None)
def _build_gather(S, NB, V, D):
    assert D == 32
    CB = 128
    assert NB // CB == _NW and NB % CB == 0
    assert S % 2 == 0
    L_rows = V // 4
    mesh = _mesh()

    GS = 33

    @functools.partial(
        pl.kernel,
        out_type=jax.ShapeDtypeStruct((S, D, NB), jnp.float32),
        mesh=mesh,
        scratch_types=[
            pltpu.VMEM((S, CB), jnp.int32),
            pltpu.VMEM((2, CB), jnp.int32),
            pltpu.VMEM((2, CB, 128), jnp.float32),
            pltpu.VMEM((2, D, CB), jnp.float32),
            pltpu.VMEM((CB * GS,), jnp.float32),
            pltpu.SemaphoreType.DMA,
            pltpu.SemaphoreType.DMA,
        ],
        compiler_params=pltpu.CompilerParams(needs_layout_passes=False),
    )
    def gather_native(tidsT, L, outT, idxall, linebuf, gbuf, obuf, tmp, gsem, osem):
        wid = lax.axis_index("s") * _NC + lax.axis_index("c")
        b0 = wid * CB
        pltpu.sync_copy(tidsT.at[:, pl.ds(b0, CB)], idxall)
        lane = lax.iota(jnp.int32, 16)
        cv = [GS * (lane + 16 * blk) for blk in range(8)]

        def prep_lines(s, par):
            for cb in range(8):
                t = idxall[s, pl.ds(cb * 16, 16)]
                linebuf[par, pl.ds(cb * 16, 16)] = jax.lax.shift_right_logical(t, 2)

        def g_copy(par):
            return pltpu.make_async_copy(
                L.at[linebuf.at[par]], gbuf.at[par], gsem
            )

        def o_copy(s, par):
            return pltpu.make_async_copy(
                obuf.at[par], outT.at[s, :, pl.ds(b0, CB)], osem
            )

        def extract(s, par):
            @plsc.parallel_loop(0, 8)
            def _(blk):
                tvec = idxall[s, pl.ds(blk * 16, 16)]
                m32v = (tvec & 3) * 32
                for i in range(16):
                    c = blk * 16 + i
                    col_lo = lane + m32v[i]
                    col_hi = col_lo + 16
                    row = jnp.full((16,), c, jnp.int32)
                    dst_lo = lane + (GS * c)
                    dst_hi = dst_lo + 16
                    v_lo = plsc.load_gather(gbuf.at[par], [row, col_lo])
                    v_hi = plsc.load_gather(gbuf.at[par], [row, col_hi])
                    plsc.store_scatter(tmp, [dst_lo], v_lo)
                    plsc.store_scatter(tmp, [dst_hi], v_hi)

            @plsc.parallel_loop(0, D, unroll=4)
            def _(d):
                for blk in range(8):
                    vals = plsc.load_gather(tmp, [cv[blk] + d])
                    obuf[par, d, pl.ds(blk * 16, 16)] = vals

        prep_lines(0, 0)
        g_copy(0).start()

        @pl.loop(0, S, step=2)
        def _(s0):
            for par in range(2):
                s = s0 + par

                @pl.when(s + 1 < S)
                def _():
                    prep_lines(s + 1, 1 - par)
                    g_copy(1 - par).start()

                g_copy(par).wait()

                @pl.when(s >= 2)
                def _():
                    o_copy(0, par).wait()

                extract(s, par)
                o_copy(s, par).start()

        o_copy(0, 0).wait()
        o_copy(0, 0).wait()

    return gather_native


def kernel(token_ids, emb):
    NB, S = token_ids.shape
    V, D = emb.shape
    embT = emb.T
    tidsT = token_ids.T.astype(jnp.int32)
    tail = V % 512
    tail_lines = emb[V - tail:, :].reshape(tail // 4, 128)
    L = _build_transpose(V, D)(embT, tail_lines)
    outT = _build_gather(S, NB, V, D)(tidsT, L)
    return outT.transpose(2, 0, 1)


# speedup vs baseline: 4.4871x; 1.0712x over previous
import functools

import jax
import jax.numpy as jnp
from jax import lax
from jax.experimental import pallas as pl
from jax.experimental.pallas import tpu as pltpu
from jax.experimental.pallas import tpu_sc as plsc

_NC = 2
_NS = 16
_NW = _NC * _NS


def _mesh():
    return plsc.VectorSubcoreMesh(
        core_axis_name="c",
        subcore_axis_name="s",
        num_cores=_NC,
        num_subcores=_NS,
    )


@functools.lru_cache(maxsize=None)
def _build_transpose(V, D):
    assert D == 32
    PW = 512
    n_full = V // PW
    tail = V % PW
    assert tail % 4 == 0 and tail < PW
    L_rows = V // 4
    NP = (n_full + _NW - 1) // _NW
    assert NP % 2 == 0
    mesh = _mesh()

    TS = PW + 1

    @functools.partial(
        pl.kernel,
        out_type=jax.ShapeDtypeStruct((L_rows, 128), jnp.float32),
        mesh=mesh,
        scratch_types=[
            pltpu.VMEM((2, D, PW), jnp.float32),
            pltpu.VMEM((2, PW // 4, 128), jnp.float32),
            pltpu.VMEM((D * TS,), jnp.float32),
            pltpu.SemaphoreType.DMA,
            pltpu.SemaphoreType.DMA,
        ],
        compiler_params=pltpu.CompilerParams(needs_layout_passes=False),
    )
    def transpose_table(embT, tail_lines, L, ibuf, obuf, tmp, isem, osem):
        wid = lax.axis_index("s") * _NC + lax.axis_index("c")
        lane = lax.iota(jnp.int32, 16)
        sd_lo = lane * TS
        sd_hi = sd_lo + 16 * TS

        def in_copy(p, b):
            return pltpu.make_async_copy(
                embT.at[:, pl.ds(p * PW, PW)], ibuf.at[b], isem
            )

        def out_copy(p, b):
            return pltpu.make_async_copy(
                obuf.at[b], L.at[pl.ds(p * (PW // 4), PW // 4), :], osem
            )

        def do_transpose(b, nlines):
            @plsc.parallel_loop(0, D, unroll=2)
            def _(d):
                base = d * TS
                for blk in range(PW // 16):
                    v = ibuf[b, d, pl.ds(blk * 16, 16)]
                    plsc.store_scatter(tmp, [lane + (base + blk * 16)], v)

            @plsc.parallel_loop(0, nlines, unroll=4)
            def _(l):
                for jblk in range(8):
                    sd = sd_lo if jblk % 2 == 0 else sd_hi
                    addr = sd + (4 * l + (jblk // 2))
                    vals = plsc.load_gather(tmp, [addr])
                    obuf[b, l, pl.ds(jblk * 16, 16)] = vals

        in_copy(wid, 0).start()

        @pl.loop(0, NP, step=2)
        def _(i0):
            for par in range(2):
                i = i0 + par
                pid = wid + _NW * i

                @pl.when(pid < n_full)
                def _():
                    in_copy(pid, par).wait()
                    nxt = pid + _NW

                    @pl.when(nxt < n_full)
                    def _():
                        in_copy(nxt, 1 - par).start()

                    @pl.when(i >= 2)
                    def _():
                        out_copy(wid, par).wait()

                    do_transpose(par, PW // 4)
                    out_copy(pid, par).start()

        out_copy(wid, 0).wait()
        out_copy(wid, 0).wait()

        if tail:
            nt = tail // 4

            @pl.when(wid == _NW - 1)
            def _():
                pltpu.sync_copy(tail_lines, obuf.at[0, pl.ds(0, nt), :])
                pltpu.sync_copy(
                    obuf.at[0, pl.ds(0, nt), :],
                    L.at[pl.ds(L_rows - nt, nt), :],
                )

    return transpose_table


@functools.lru_cache(maxsize=None)
def _build_gather(S, NB, V, D):
    assert D == 32
    CB = 128
    assert NB // CB == _NW and NB % CB == 0
    assert S % 2 == 0
    L_rows = V // 4
    mesh = _mesh()

    GS = 33

    @functools.partial(
        pl.kernel,
        out_type=jax.ShapeDtypeStruct((S, D, NB), jnp.float32),
        mesh=mesh,
        scratch_types=[
            pltpu.VMEM((S, CB), jnp.int32),
            pltpu.VMEM((4, CB), jnp.int32),
            pltpu.VMEM((4, CB, 128), jnp.float32),
            pltpu.VMEM((2, D, CB), jnp.float32),
            pltpu.VMEM((CB * GS,), jnp.float32),
            pltpu.SemaphoreType.DMA,
            pltpu.SemaphoreType.DMA,
        ],
        compiler_params=pltpu.CompilerParams(needs_layout_passes=False),
    )
    def gather_native(tidsT, L, outT, idxall, linebuf, gbuf, obuf, tmp, gsem, osem):
        wid = lax.axis_index("s") * _NC + lax.axis_index("c")
        b0 = wid * CB
        pltpu.sync_copy(tidsT.at[:, pl.ds(b0, CB)], idxall)
        lane = lax.iota(jnp.int32, 16)
        cv = [GS * (lane + 16 * blk) for blk in range(8)]

        def prep_lines(s, par):
            for cb in range(8):
                t = idxall[s, pl.ds(cb * 16, 16)]
                linebuf[par, pl.ds(cb * 16, 16)] = jax.lax.shift_right_logical(t, 2)

        def g_copy(par):
            return pltpu.make_async_copy(
                L.at[linebuf.at[par]], gbuf.at[par], gsem
            )

        def o_copy(s, par):
            return pltpu.make_async_copy(
                obuf.at[par], outT.at[s, :, pl.ds(b0, CB)], osem
            )

        def extract(s, gq, oq):
            @plsc.parallel_loop(0, 8)
            def _(blk):
                tvec = idxall[s, pl.ds(blk * 16, 16)]
                m32v = (tvec & 3) * 32
                for i in range(16):
                    c = blk * 16 + i
                    col_lo = lane + m32v[i]
                    col_hi = col_lo + 16
                    row = jnp.full((16,), c, jnp.int32)
                    dst_lo = lane + (GS * c)
                    dst_hi = dst_lo + 16
                    v_lo = plsc.load_gather(gbuf.at[gq], [row, col_lo])
                    v_hi = plsc.load_gather(gbuf.at[gq], [row, col_hi])
                    plsc.store_scatter(tmp, [dst_lo], v_lo)
                    plsc.store_scatter(tmp, [dst_hi], v_hi)

            @plsc.parallel_loop(0, D, unroll=4)
            def _(d):
                for blk in range(8):
                    vals = plsc.load_gather(tmp, [cv[blk] + d])
                    obuf[oq, d, pl.ds(blk * 16, 16)] = vals

        for r in range(3):
            prep_lines(r, r)
            g_copy(r).start()

        @pl.loop(0, S, step=4)
        def _(s0):
            for q in range(4):
                s = s0 + q
                nxt = (q + 3) % 4

                @pl.when(s + 3 < S)
                def _():
                    prep_lines(s + 3, nxt)
                    g_copy(nxt).start()

                g_copy(q).wait()

                @pl.when(s >= 2)
                def _():
                    o_copy(0, q % 2).wait()

                extract(s, q, q % 2)
                o_copy(s, q % 2).start()

        o_copy(0, 0).wait()
        o_copy(0, 0).wait()

    return gather_native


def kernel(token_ids, emb):
    NB, S = token_ids.shape
    V, D = emb.shape
    embT = emb.T
    tidsT = token_ids.T.astype(jnp.int32)
    tail = V % 512
    tail_lines = emb[V - tail:, :].reshape(tail // 4, 128)
    L = _build_transpose(V, D)(embT, tail_lines)
    outT = _build_gather(S, NB, V, D)(tidsT, L)
    return outT.transpose(2, 0, 1)
